# baseline (speedup 1.0000x reference)
import jax
import jax.numpy as jnp
from jax import lax
from jax.experimental import pallas as pl


def kernel(input, table):
    B, S, D = input.shape
    V = table.shape[0]
    F = S * D
    BB = 128

    tbl2 = jnp.reshape(table, (1, V * D))

    def body(t_ref, out_ref):
        emb = t_ref[:, :F]
        out_ref[...] = jnp.broadcast_to(emb, (BB, F))

    out2 = pl.pallas_call(
        body,
        grid=(B // BB,),
        in_specs=[pl.BlockSpec((1, V * D), lambda i: (0, 0))],
        out_specs=pl.BlockSpec((BB, F), lambda i: (i, 0)),
        out_shape=jax.ShapeDtypeStruct((B, F), jnp.float32),
    )(tbl2)
    out2 = lax.optimization_barrier(out2)
    return jnp.reshape(out2, (B, S, D))
